# baseline (speedup 1.0000x reference)
import functools

import jax
import jax.numpy as jnp
from jax import lax
from jax.experimental import pallas as pl
from jax.experimental.pallas import tpu as pltpu
from jax.experimental.pallas import tpu_sc as plsc

N = 10000
E = 320000
D = 128
R = 3
H = 64

TILES = 16
SUB = 128
NB16 = 4
NGIF = 3
HCH = 1024
NSUB = HCH // SUB
EP = 327680
EPT = EP // TILES
NSUPER = EPT // HCH
NP = 10240
ZPT = NP // TILES
TBL = 3 * N


@functools.partial(
    pl.kernel,
    mesh=plsc.VectorSubcoreMesh(core_axis_name="c", subcore_axis_name="s"),
    out_type=jax.ShapeDtypeStruct((2 * NP, H), jnp.float32),
    compiler_params=pltpu.CompilerParams(use_tc_tiling_on_sc=False),
    scratch_types=[
        pltpu.VMEM((NSUB, SUB), jnp.int32),
        pltpu.VMEM((NSUB, SUB), jnp.int32),
        pltpu.VMEM((HCH,), jnp.float32),
        pltpu.VMEM((SUB, H // 2), jnp.int32),
        pltpu.VMEM((SUB, H // 2), jnp.int32),
        pltpu.VMEM((SUB, H // 2), jnp.int32),
        pltpu.VMEM((SUB, H // 2), jnp.int32),
        pltpu.VMEM((SUB, H), jnp.float32),
        pltpu.VMEM((SUB, H), jnp.float32),
        pltpu.VMEM_SHARED((NP, H), jnp.float32),
        pltpu.SemaphoreType.DMA,
        pltpu.SemaphoreType.DMA,
        pltpu.SemaphoreType.DMA,
        pltpu.SemaphoreType.DMA,
        pltpu.SemaphoreType.DMA,
        pltpu.SemaphoreType.DMA,
        pltpu.SemaphoreType.DMA,
    ],
)
def _sc_edge_accum(tbl, gidx2, dst2, w2, out,
                   gidx_v, dst_v, w_v, g0, g1, g2, g3, f0, f1, acc,
                   gs0, gs1, gs2, gs3, ss0, ss1, isem):
    c = lax.axis_index("c")
    s = lax.axis_index("s")
    gbufs = ((g0, gs0), (g1, gs1), (g2, gs2), (g3, gs3))
    fbufs = ((f0, ss0), (f1, ss1))

    zero = jnp.zeros((16,), jnp.float32)

    def _zrow(i, carry):
        for u in range(H // 16):
            f0[i, pl.ds(u * 16, 16)] = zero
        return carry

    lax.fori_loop(0, SUB, _zrow, 0)

    zb = s * ZPT

    def _zacc(q, carry):
        pltpu.sync_copy(f0, acc.at[pl.ds(zb + q * SUB, SUB)])
        return carry

    lax.fori_loop(0, ZPT // SUB, _zacc, 0)
    plsc.subcore_barrier()

    def _super(k, carry):
        rb = s * (EPT // SUB) + k * NSUB
        i0 = pltpu.async_copy(dst2.at[pl.ds(rb, NSUB)], dst_v, isem)
        i1 = pltpu.async_copy(gidx2.at[pl.ds(c * (EP // SUB) + rb, NSUB)],
                              gidx_v, isem)
        i2 = pltpu.async_copy(w2.at[pl.ds(s * EPT + k * HCH, HCH)],
                              w_v, isem)
        i0.wait()
        i1.wait()
        i2.wait()

        gat = {}
        sca = {}
        for j in range(NGIF):
            gat[j] = pltpu.async_copy(tbl.at[gidx_v.at[j]],
                                      gbufs[j % NB16][0], gbufs[j % NB16][1])
        for j in range(NSUB):
            gbuf, _ = gbufs[j % NB16]
            fbuf, fsem = fbufs[j % 2]
            if j + NGIF < NSUB:
                nbuf, ngsem = gbufs[(j + NGIF) % NB16]
                gat[j + NGIF] = pltpu.async_copy(
                    tbl.at[gidx_v.at[j + NGIF]], nbuf, ngsem)
            if j - 2 >= 0:
                sca[j - 2].wait()
            gat[j].wait()

            def _sgrp(g, cc, gbuf=gbuf, fbuf=fbuf, j=j):
                w16 = w_v[pl.ds(j * SUB + g * 16, 16)]
                himask = jnp.full((16,), -65536, jnp.int32)
                for t in range(16):
                    wt = w16[t]
                    row = g * 16 + t
                    for u in range(H // 32):
                        xi = gbuf[row, pl.ds(u * 16, 16)]
                        lo = lax.bitcast_convert_type(xi << 16, jnp.float32)
                        hi = lax.bitcast_convert_type(xi & himask, jnp.float32)
                        fbuf[row, pl.ds(u * 32, 16)] = lo * wt
                        fbuf[row, pl.ds(u * 32 + 16, 16)] = hi * wt
                return cc

            lax.fori_loop(0, SUB // 16, _sgrp, 0)

            sca[j] = pltpu.async_copy(fbuf, acc.at[dst_v.at[j]], fsem,
                                      add=True)
        sca[NSUB - 2].wait()
        sca[NSUB - 1].wait()
        return carry

    lax.fori_loop(0, NSUPER, _super, 0)
    plsc.subcore_barrier()

    pltpu.sync_copy(acc.at[pl.ds(s * ZPT, ZPT)],
                    out.at[pl.ds(c * NP + s * ZPT, ZPT)])


def _tc_pre0_body(x_ref, wrel_ref, wroot_ref, wskip_ref, b_ref, g_ref, d_ref):
    h = x_ref[...]
    for r in range(R):
        gr = jnp.dot(h, wrel_ref[r], preferred_element_type=jnp.float32)
        g_ref[0, r] = gr[:, :H]
        g_ref[1, r] = gr[:, H:]
    d_ref[...] = (jnp.dot(h, wroot_ref[...], preferred_element_type=jnp.float32)
                  + jnp.dot(h, wskip_ref[...], preferred_element_type=jnp.float32)
                  + b_ref[...])


def _tc_pre1_body(d_ref, cv_ref, x_ref, wrel_ref, wroot_ref, wskip_ref, b_ref,
                  g_ref, dout_ref):
    h = d_ref[...] + jnp.concatenate([cv_ref[0], cv_ref[1]], axis=1)
    for r in range(R):
        gr = jnp.dot(h, wrel_ref[r], preferred_element_type=jnp.float32)
        g_ref[0, r] = gr[:, :H]
        g_ref[1, r] = gr[:, H:]
    dout_ref[...] = (jnp.dot(h, wroot_ref[...], preferred_element_type=jnp.float32)
                     + jnp.dot(x_ref[...], wskip_ref[...],
                               preferred_element_type=jnp.float32)
                     + b_ref[...])


def _tc_fin_body(d_ref, cv_ref, out_ref):
    out_ref[...] = d_ref[...] + jnp.concatenate([cv_ref[0], cv_ref[1]], axis=1)


_BLK = 1000


def _mat_spec():
    return pl.BlockSpec((D, D), lambda i: (0, 0))


def _rows_spec():
    return pl.BlockSpec((_BLK, D), lambda i: (i, 0))


def _conv_spec():
    return pl.BlockSpec((2, _BLK, H), lambda i: (0, i, 0))


def _tc_pre0(x, wrel, wroot, wskip, b):
    return pl.pallas_call(
        _tc_pre0_body,
        grid=(N // _BLK,),
        in_specs=[_rows_spec(),
                  pl.BlockSpec((R, D, D), lambda i: (0, 0, 0)),
                  _mat_spec(), _mat_spec(),
                  pl.BlockSpec((1, D), lambda i: (0, 0))],
        out_specs=[pl.BlockSpec((2, R, _BLK, H), lambda i: (0, 0, i, 0)),
                   _rows_spec()],
        out_shape=[jax.ShapeDtypeStruct((2, R, N, H), jnp.float32),
                   jax.ShapeDtypeStruct((N, D), jnp.float32)],
    )(x, wrel, wroot, wskip, b)


def _tc_pre1(d, cv, x, wrel, wroot, wskip, b):
    return pl.pallas_call(
        _tc_pre1_body,
        grid=(N // _BLK,),
        in_specs=[_rows_spec(), _conv_spec(), _rows_spec(),
                  pl.BlockSpec((R, D, D), lambda i: (0, 0, 0)),
                  _mat_spec(), _mat_spec(),
                  pl.BlockSpec((1, D), lambda i: (0, 0))],
        out_specs=[pl.BlockSpec((2, R, _BLK, H), lambda i: (0, 0, i, 0)),
                   _rows_spec()],
        out_shape=[jax.ShapeDtypeStruct((2, R, N, H), jnp.float32),
                   jax.ShapeDtypeStruct((N, D), jnp.float32)],
    )(d, cv, x, wrel, wroot, wskip, b)


def _tc_fin(d, cv):
    return pl.pallas_call(
        _tc_fin_body,
        grid=(N // _BLK,),
        in_specs=[_rows_spec(), _conv_spec()],
        out_specs=_rows_spec(),
        out_shape=jax.ShapeDtypeStruct((N, D), jnp.float32),
    )(d, cv)


def _pack_table(g):
    gb = jax.lax.bitcast_convert_type(
        g.astype(jnp.bfloat16), jnp.uint16).astype(jnp.int32)
    gb = gb.reshape(2, R, N, H // 32, 2, 16)
    packed = gb[..., 0, :] | (gb[..., 1, :] << 16)
    return packed.reshape(2 * R * N, H // 2)


def kernel(x, edge_index, edge_type, edge_weight,
           W_rel0, W_root0, b_conv0, W_skip0, b_skip0,
           W_rel1, W_root1, b_conv1, W_skip1, b_skip1):
    src = edge_index[0]
    dst = edge_index[1]
    gidx = edge_type * N + src

    pad = EP - E
    gidx_p = jnp.pad(gidx, (0, pad))
    gidx2 = jnp.concatenate([gidx_p, gidx_p + TBL]).reshape(2 * EP // SUB, SUB)
    dst2 = jnp.pad(dst, (0, pad)).reshape(EP // SUB, SUB)
    w2 = jnp.pad(edge_weight, (0, pad))

    b0 = (b_conv0 + b_skip0).reshape(1, D)
    b1 = (b_conv1 + b_skip1).reshape(1, D)

    g0, d0 = _tc_pre0(x, W_rel0, W_root0, W_skip0, b0)
    cv0 = _sc_edge_accum(_pack_table(g0), gidx2, dst2, w2)
    cv0 = cv0.reshape(2, NP, H)[:, :N, :]
    g1, d1 = _tc_pre1(d0, cv0, x, W_rel1, W_root1, W_skip1, b1)
    cv1 = _sc_edge_accum(_pack_table(g1), gidx2, dst2, w2)
    cv1 = cv1.reshape(2, NP, H)[:, :N, :]

    return _tc_fin(d1, cv1)
